# baseline (speedup 1.0000x reference)
import functools

import jax
import jax.numpy as jnp
from jax import lax
from jax.experimental import pallas as pl
from jax.experimental.pallas import tpu as pltpu
from jax.experimental.pallas import tpu_sc as plsc

N = 10000
E = 320000
D = 128

NC = 2
NS = 16
NW = NC * NS
EPW = E // NW
K = 40
NCHUNK = EPW // K
NSET = 3
RPT = 624
TAIL = N - NS * RPT

BM = 1000
GRID = N // BM


def _pre_body(x_ref, wes_ref, wed_ref, be_ref, w1_ref, b1_ref,
              a_ref, b_ref, t0_ref):
    x = x_ref[...]
    a_ref[...] = jnp.dot(x, wes_ref[...],
                         preferred_element_type=jnp.float32)
    b_ref[...] = jnp.dot(x, wed_ref[...],
                         preferred_element_type=jnp.float32) + be_ref[...]
    t0_ref[...] = jnp.dot(x, w1_ref[...],
                          preferred_element_type=jnp.float32) + b1_ref[...]


def _pre(x, wes_t, wed_t, be, w1_t, b1):
    row = pl.BlockSpec((BM, D), lambda i: (i, 0))
    wspec = pl.BlockSpec((D, D), lambda i: (0, 0))
    bspec = pl.BlockSpec((1, D), lambda i: (0, 0))
    return pl.pallas_call(
        _pre_body,
        grid=(GRID,),
        in_specs=[row, wspec, wspec, bspec, wspec, bspec],
        out_specs=[row, row, row],
        out_shape=[jax.ShapeDtypeStruct((N, D), jnp.float32)] * 3,
    )(x, wes_t, wed_t, be.reshape(1, D), w1_t, b1.reshape(1, D))


def _edge_agg(src, dst, a_tab, b_tab):
    mesh = plsc.VectorSubcoreMesh(core_axis_name="c", subcore_axis_name="s")

    @functools.partial(
        pl.kernel,
        mesh=mesh,
        out_type=jax.ShapeDtypeStruct((NC, N, D), jnp.float32),
        scratch_types=(
            [pltpu.VMEM((K,), jnp.int32)] * (3 * NSET)
            + [pltpu.VMEM((K, D), jnp.float32)] * (3 * NSET)
            + [pltpu.VMEM_SHARED((N, D), jnp.float32)]
            + [pltpu.SemaphoreType.DMA] * (3 * NSET)
        ),
    )
    def k(src_hbm, dst_hbm, a_hbm, b_hbm, out_hbm,
          si0, si1, si2, di0, di1, di2, sd0, sd1, sd2,
          a0, a1, a2, b0, b1, b2, m0, m1, m2, agg_sh,
          gsem0, gsem1, gsem2, isem0, isem1, isem2,
          ssem0, ssem1, ssem2):
        cid = lax.axis_index("c")
        sid = lax.axis_index("s")
        wid = cid * NS + sid
        sibuf = (si0, si1, si2)
        dibuf = (di0, di1, di2)
        sdbuf = (sd0, sd1, sd2)
        abuf = (a0, a1, a2)
        bbuf = (b0, b1, b2)
        mbuf = (m0, m1, m2)
        gsem = (gsem0, gsem1, gsem2)
        isem = (isem0, isem1, isem2)
        ssem = (ssem0, ssem1, ssem2)
        ebase = wid * EPW

        zero = jnp.zeros((16,), jnp.float32)

        def zrow(j, carry):
            for r in range(8):
                m0[j, pl.ds(r * 16, 16)] = zero
            return carry

        lax.fori_loop(0, K, zrow, 0)

        base_row = sid * RPT
        for r in range(RPT // K):
            pltpu.sync_copy(m0, agg_sh.at[pl.ds(base_row + r * K, K)])
        if RPT % K:
            pltpu.sync_copy(
                m0.at[pl.ds(0, RPT % K)],
                agg_sh.at[pl.ds(base_row + (RPT // K) * K, RPT % K)])

        @pl.when(sid == NS - 1)
        def _():
            pltpu.sync_copy(m0.at[pl.ds(0, TAIL)],
                            agg_sh.at[pl.ds(NS * RPT, TAIL)])

        plsc.subcore_barrier()

        def fetch_idx(c, s):
            off = ebase + c * K
            pltpu.async_copy(src_hbm.at[pl.ds(off, K)], sibuf[s], isem[s])
            pltpu.async_copy(dst_hbm.at[pl.ds(off, K)], dibuf[s], isem[s])

        def wait_idx(s):
            pltpu.make_async_copy(src_hbm.at[pl.ds(0, K)], sibuf[s],
                                  isem[s]).wait()
            pltpu.make_async_copy(dst_hbm.at[pl.ds(0, K)], dibuf[s],
                                  isem[s]).wait()

        def gathers(s):
            pltpu.async_copy(a_hbm.at[sibuf[s]], abuf[s], gsem[s])
            pltpu.async_copy(b_hbm.at[dibuf[s]], bbuf[s], gsem[s])

        def wait_gathers(s):
            pltpu.make_async_copy(a_hbm.at[sibuf[s]], abuf[s],
                                  gsem[s]).wait()
            pltpu.make_async_copy(b_hbm.at[dibuf[s]], bbuf[s],
                                  gsem[s]).wait()

        def compute(s):
            a_v, b_v, m_v = abuf[s], bbuf[s], mbuf[s]

            @plsc.parallel_loop(0, K, 1, unroll=4)
            def _row(j):
                for r in range(8):
                    sl = pl.ds(r * 16, 16)
                    m_v[j, sl] = jnp.maximum(a_v[j, sl] + b_v[j, sl], 0.0)
            offs = list(range(0, K - 15, 16))
            if K % 16:
                offs.append(K - 16)
            for off in offs:
                sl = pl.ds(off, 16)
                sdbuf[s][sl] = dibuf[s][sl]

        def scatter(s):
            pltpu.async_copy(mbuf[s], agg_sh.at[sdbuf[s]], ssem[s],
                             add=True)

        def wait_scatter(s):
            pltpu.make_async_copy(mbuf[s], agg_sh.at[sdbuf[s]],
                                  ssem[s]).wait()

        fetch_idx(0, 0)
        fetch_idx(1, 1)
        fetch_idx(2, 2)
        wait_idx(0)
        gathers(0)
        wait_idx(1)
        gathers(1)

        def step(i, cc, s):
            s2 = (s + 2) % NSET

            @pl.when(cc + 2 < NCHUNK)
            def _():
                wait_idx(s2)
                gathers(s2)

            wait_gathers(s)

            @pl.when(i > 0)
            def _():
                wait_scatter(s)

            compute(s)
            scatter(s)

            @pl.when(cc + NSET < NCHUNK)
            def _():
                fetch_idx(cc + NSET, s)

        def triple(i, carry):
            c = NSET * i
            for t in range(NSET):
                step(i, c + t, t)
            return carry

        lax.fori_loop(0, NCHUNK // NSET, triple, 0)
        wait_gathers(0)
        wait_scatter(0)
        compute(0)
        scatter(0)
        wait_scatter(1)
        wait_scatter(2)
        wait_scatter(0)
        plsc.subcore_barrier()
        pltpu.sync_copy(agg_sh.at[pl.ds(base_row, RPT)],
                        out_hbm.at[cid, pl.ds(base_row, RPT)])

        @pl.when(sid == NS - 1)
        def _():
            pltpu.sync_copy(agg_sh.at[pl.ds(NS * RPT, TAIL)],
                            out_hbm.at[cid, pl.ds(NS * RPT, TAIL)])

    return k(src, dst, a_tab, b_tab)


def _tail_body(agg_ref, t0_ref, wn_ref, bn_ref, w2_ref, b2_ref,
               wl_ref, bl_ref, rm_ref, rv_ref, g_ref, bb_ref,
               w4_ref, b4_ref, out_ref):
    agg = agg_ref[0] + agg_ref[1]
    h = jnp.maximum(
        jnp.dot(agg, wn_ref[...],
                preferred_element_type=jnp.float32) + bn_ref[...], 0.0)
    t = t0_ref[...] + jnp.dot(h, w2_ref[...],
                              preferred_element_type=jnp.float32) + b2_ref[...]
    for j in range(4):
        z = jnp.dot(t, wl_ref[j],
                    preferred_element_type=jnp.float32) + bl_ref[j]
        scale = jax.lax.rsqrt(rv_ref[j] + 1e-5) * g_ref[j]
        t = jnp.maximum((z - rm_ref[j]) * scale + bb_ref[j], 0.0)
    out_ref[...] = jnp.dot(t, w4_ref[...],
                           preferred_element_type=jnp.float32) + b4_ref[...]


def _tail(aggp, t0, wn_t, bn, w2_t, b2, wl_t, bl, rm, rv, g, bb, w4_t, b4):
    row = pl.BlockSpec((BM, D), lambda i: (i, 0))
    aggspec = pl.BlockSpec((NC, BM, D), lambda i: (0, i, 0))
    wspec = pl.BlockSpec((D, D), lambda i: (0, 0))
    bspec = pl.BlockSpec((1, D), lambda i: (0, 0))
    wlspec = pl.BlockSpec((4, D, D), lambda i: (0, 0, 0))
    blspec = pl.BlockSpec((4, 1, D), lambda i: (0, 0, 0))
    return pl.pallas_call(
        _tail_body,
        grid=(GRID,),
        in_specs=[aggspec, row, wspec, bspec, wspec, bspec,
                  wlspec, blspec, blspec, blspec, blspec, blspec,
                  wspec, bspec],
        out_specs=row,
        out_shape=jax.ShapeDtypeStruct((N, D), jnp.float32),
    )(aggp, t0, wn_t, bn.reshape(1, D), w2_t, b2.reshape(1, D),
      wl_t, bl.reshape(4, 1, D), rm.reshape(4, 1, D), rv.reshape(4, 1, D),
      g.reshape(4, 1, D), bb.reshape(4, 1, D), w4_t, b4.reshape(1, D))


def kernel(L_x_, L_edge_index_, L_self_modules_edge_lin_parameters_weight_, L_self_modules_edge_lin_parameters_bias_, L_self_modules_cat_lin1_parameters_weight_, L_self_modules_cat_lin1_parameters_bias_, L_self_modules_cat_lin2_parameters_weight_, L_self_modules_cat_lin2_parameters_bias_, L_self_modules_node_mlp_modules_lins_modules_0_parameters_weight_, L_self_modules_node_mlp_modules_lins_modules_0_parameters_bias_, L_self_modules_final_mlp_modules_lins_modules_0_parameters_weight_, L_self_modules_final_mlp_modules_lins_modules_0_parameters_bias_, L_self_modules_final_mlp_modules_lins_modules_1_parameters_weight_, L_self_modules_final_mlp_modules_lins_modules_1_parameters_bias_, L_self_modules_final_mlp_modules_lins_modules_2_parameters_weight_, L_self_modules_final_mlp_modules_lins_modules_2_parameters_bias_, L_self_modules_final_mlp_modules_lins_modules_3_parameters_weight_, L_self_modules_final_mlp_modules_lins_modules_3_parameters_bias_, L_self_modules_final_mlp_modules_lins_modules_4_parameters_weight_, L_self_modules_final_mlp_modules_lins_modules_4_parameters_bias_, L_self_modules_final_mlp_modules_norms_modules_0_modules_module_buffers_running_mean_, L_self_modules_final_mlp_modules_norms_modules_0_modules_module_buffers_running_var_, L_self_modules_final_mlp_modules_norms_modules_0_modules_module_parameters_weight_, L_self_modules_final_mlp_modules_norms_modules_0_modules_module_parameters_bias_, L_self_modules_final_mlp_modules_norms_modules_1_modules_module_buffers_running_mean_, L_self_modules_final_mlp_modules_norms_modules_1_modules_module_buffers_running_var_, L_self_modules_final_mlp_modules_norms_modules_1_modules_module_parameters_weight_, L_self_modules_final_mlp_modules_norms_modules_1_modules_module_parameters_bias_, L_self_modules_final_mlp_modules_norms_modules_2_modules_module_buffers_running_mean_, L_self_modules_final_mlp_modules_norms_modules_2_modules_module_buffers_running_var_, L_self_modules_final_mlp_modules_norms_modules_2_modules_module_parameters_weight_, L_self_modules_final_mlp_modules_norms_modules_2_modules_module_parameters_bias_, L_self_modules_final_mlp_modules_norms_modules_3_modules_module_buffers_running_mean_, L_self_modules_final_mlp_modules_norms_modules_3_modules_module_buffers_running_var_, L_self_modules_final_mlp_modules_norms_modules_3_modules_module_parameters_weight_, L_self_modules_final_mlp_modules_norms_modules_3_modules_module_parameters_bias_):
    x = L_x_
    ei = L_edge_index_
    src = ei[0].astype(jnp.int32)
    dst = ei[1].astype(jnp.int32)
    we = L_self_modules_edge_lin_parameters_weight_
    be = L_self_modules_edge_lin_parameters_bias_

    a_tab, b_tab, t0 = _pre(
        x, we[:, :D].T, we[:, D:].T, be,
        L_self_modules_cat_lin1_parameters_weight_.T,
        L_self_modules_cat_lin1_parameters_bias_)

    aggp = _edge_agg(src, dst, a_tab, b_tab)

    wl_t = jnp.stack([
        L_self_modules_final_mlp_modules_lins_modules_0_parameters_weight_.T,
        L_self_modules_final_mlp_modules_lins_modules_1_parameters_weight_.T,
        L_self_modules_final_mlp_modules_lins_modules_2_parameters_weight_.T,
        L_self_modules_final_mlp_modules_lins_modules_3_parameters_weight_.T,
    ])
    bl = jnp.stack([
        L_self_modules_final_mlp_modules_lins_modules_0_parameters_bias_,
        L_self_modules_final_mlp_modules_lins_modules_1_parameters_bias_,
        L_self_modules_final_mlp_modules_lins_modules_2_parameters_bias_,
        L_self_modules_final_mlp_modules_lins_modules_3_parameters_bias_,
    ])
    rm = jnp.stack([
        L_self_modules_final_mlp_modules_norms_modules_0_modules_module_buffers_running_mean_,
        L_self_modules_final_mlp_modules_norms_modules_1_modules_module_buffers_running_mean_,
        L_self_modules_final_mlp_modules_norms_modules_2_modules_module_buffers_running_mean_,
        L_self_modules_final_mlp_modules_norms_modules_3_modules_module_buffers_running_mean_,
    ])
    rv = jnp.stack([
        L_self_modules_final_mlp_modules_norms_modules_0_modules_module_buffers_running_var_,
        L_self_modules_final_mlp_modules_norms_modules_1_modules_module_buffers_running_var_,
        L_self_modules_final_mlp_modules_norms_modules_2_modules_module_buffers_running_var_,
        L_self_modules_final_mlp_modules_norms_modules_3_modules_module_buffers_running_var_,
    ])
    g = jnp.stack([
        L_self_modules_final_mlp_modules_norms_modules_0_modules_module_parameters_weight_,
        L_self_modules_final_mlp_modules_norms_modules_1_modules_module_parameters_weight_,
        L_self_modules_final_mlp_modules_norms_modules_2_modules_module_parameters_weight_,
        L_self_modules_final_mlp_modules_norms_modules_3_modules_module_parameters_weight_,
    ])
    bb = jnp.stack([
        L_self_modules_final_mlp_modules_norms_modules_0_modules_module_parameters_bias_,
        L_self_modules_final_mlp_modules_norms_modules_1_modules_module_parameters_bias_,
        L_self_modules_final_mlp_modules_norms_modules_2_modules_module_parameters_bias_,
        L_self_modules_final_mlp_modules_norms_modules_3_modules_module_parameters_bias_,
    ])

    return _tail(
        aggp, t0,
        L_self_modules_node_mlp_modules_lins_modules_0_parameters_weight_.T,
        L_self_modules_node_mlp_modules_lins_modules_0_parameters_bias_,
        L_self_modules_cat_lin2_parameters_weight_.T,
        L_self_modules_cat_lin2_parameters_bias_,
        wl_t, bl, rm, rv, g, bb,
        L_self_modules_final_mlp_modules_lins_modules_4_parameters_weight_.T,
        L_self_modules_final_mlp_modules_lins_modules_4_parameters_bias_)


# speedup vs baseline: 1.0129x; 1.0129x over previous
import functools

import jax
import jax.numpy as jnp
from jax import lax
from jax.experimental import pallas as pl
from jax.experimental.pallas import tpu as pltpu
from jax.experimental.pallas import tpu_sc as plsc

N = 10000
E = 320000
D = 128

NC = 2
NS = 16
NW = NC * NS
EPW = E // NW
K = 40
NCHUNK = EPW // K
NSET = 3
RPT = 624
TAIL = N - NS * RPT

BM = 1000
GRID = N // BM


def _dot_nt(x, w):
    return lax.dot_general(x, w, (((1,), (1,)), ((), ())),
                           preferred_element_type=jnp.float32)


def _pre_body(x_ref, wes_ref, wed_ref, be_ref, w1_ref, b1_ref,
              a_ref, b_ref, t0_ref):
    x = x_ref[...]
    a_ref[...] = _dot_nt(x, wes_ref[...])
    b_ref[...] = _dot_nt(x, wed_ref[...]) + be_ref[...]
    t0_ref[...] = _dot_nt(x, w1_ref[...]) + b1_ref[...]


def _pre(x, we, be, w1, b1):
    row = pl.BlockSpec((BM, D), lambda i: (i, 0))
    wspec = pl.BlockSpec((D, D), lambda i: (0, 0))
    bspec = pl.BlockSpec((1, D), lambda i: (0, 0))
    wes = pl.BlockSpec((D, D), lambda i: (0, 0))
    wed = pl.BlockSpec((D, D), lambda i: (0, 1))
    return pl.pallas_call(
        _pre_body,
        grid=(GRID,),
        in_specs=[row, wes, wed, bspec, wspec, bspec],
        out_specs=[row, row, row],
        out_shape=[jax.ShapeDtypeStruct((N, D), jnp.float32)] * 3,
    )(x, we, we, be.reshape(1, D), w1, b1.reshape(1, D))


def _edge_agg(src, dst, a_tab, b_tab):
    mesh = plsc.VectorSubcoreMesh(core_axis_name="c", subcore_axis_name="s")

    @functools.partial(
        pl.kernel,
        mesh=mesh,
        out_type=jax.ShapeDtypeStruct((NC, N, D), jnp.float32),
        scratch_types=(
            [pltpu.VMEM((K,), jnp.int32)] * (3 * NSET)
            + [pltpu.VMEM((K, D), jnp.float32)] * (3 * NSET)
            + [pltpu.VMEM_SHARED((N, D), jnp.float32)]
            + [pltpu.SemaphoreType.DMA] * (3 * NSET)
        ),
    )
    def k(src_hbm, dst_hbm, a_hbm, b_hbm, out_hbm,
          si0, si1, si2, di0, di1, di2, sd0, sd1, sd2,
          a0, a1, a2, b0, b1, b2, m0, m1, m2, agg_sh,
          gsem0, gsem1, gsem2, isem0, isem1, isem2,
          ssem0, ssem1, ssem2):
        cid = lax.axis_index("c")
        sid = lax.axis_index("s")
        wid = cid * NS + sid
        sibuf = (si0, si1, si2)
        dibuf = (di0, di1, di2)
        sdbuf = (sd0, sd1, sd2)
        abuf = (a0, a1, a2)
        bbuf = (b0, b1, b2)
        mbuf = (m0, m1, m2)
        gsem = (gsem0, gsem1, gsem2)
        isem = (isem0, isem1, isem2)
        ssem = (ssem0, ssem1, ssem2)
        ebase = wid * EPW

        def fetch_idx(c, s):
            off = ebase + c * K
            pltpu.async_copy(src_hbm.at[pl.ds(off, K)], sibuf[s], isem[s])
            pltpu.async_copy(dst_hbm.at[pl.ds(off, K)], dibuf[s], isem[s])

        def wait_idx(s):
            pltpu.make_async_copy(src_hbm.at[pl.ds(0, K)], sibuf[s],
                                  isem[s]).wait()
            pltpu.make_async_copy(dst_hbm.at[pl.ds(0, K)], dibuf[s],
                                  isem[s]).wait()

        def gathers(s):
            pltpu.async_copy(a_hbm.at[sibuf[s]], abuf[s], gsem[s])
            pltpu.async_copy(b_hbm.at[dibuf[s]], bbuf[s], gsem[s])

        def wait_gathers(s):
            pltpu.make_async_copy(a_hbm.at[sibuf[s]], abuf[s],
                                  gsem[s]).wait()
            pltpu.make_async_copy(b_hbm.at[dibuf[s]], bbuf[s],
                                  gsem[s]).wait()

        def compute(s):
            a_v, b_v, m_v = abuf[s], bbuf[s], mbuf[s]

            @plsc.parallel_loop(0, K, 1, unroll=4)
            def _row(j):
                for r in range(8):
                    sl = pl.ds(r * 16, 16)
                    m_v[j, sl] = jnp.maximum(a_v[j, sl] + b_v[j, sl], 0.0)
            offs = list(range(0, K - 15, 16))
            if K % 16:
                offs.append(K - 16)
            for off in offs:
                sl = pl.ds(off, 16)
                sdbuf[s][sl] = dibuf[s][sl]

        def scatter(s):
            pltpu.async_copy(mbuf[s], agg_sh.at[sdbuf[s]], ssem[s],
                             add=True)

        def wait_scatter(s):
            pltpu.make_async_copy(mbuf[s], agg_sh.at[sdbuf[s]],
                                  ssem[s]).wait()

        fetch_idx(0, 0)
        fetch_idx(1, 1)
        fetch_idx(2, 2)
        wait_idx(0)
        gathers(0)
        wait_idx(1)
        gathers(1)

        zero = jnp.zeros((16,), jnp.float32)

        def zrow(j, carry):
            for r in range(8):
                m0[j, pl.ds(r * 16, 16)] = zero
            return carry

        lax.fori_loop(0, K, zrow, 0)

        base_row = sid * RPT
        for r in range(RPT // K):
            pltpu.sync_copy(m0, agg_sh.at[pl.ds(base_row + r * K, K)])
        if RPT % K:
            pltpu.sync_copy(
                m0.at[pl.ds(0, RPT % K)],
                agg_sh.at[pl.ds(base_row + (RPT // K) * K, RPT % K)])

        @pl.when(sid == NS - 1)
        def _():
            pltpu.sync_copy(m0.at[pl.ds(0, TAIL)],
                            agg_sh.at[pl.ds(NS * RPT, TAIL)])

        plsc.subcore_barrier()

        def step(i, cc, s):
            s2 = (s + 2) % NSET

            @pl.when(cc + 2 < NCHUNK)
            def _():
                wait_idx(s2)
                gathers(s2)

            wait_gathers(s)

            @pl.when(i > 0)
            def _():
                wait_scatter(s)

            compute(s)
            scatter(s)

            @pl.when(cc + NSET < NCHUNK)
            def _():
                fetch_idx(cc + NSET, s)

        def triple(i, carry):
            c = NSET * i
            for t in range(NSET):
                step(i, c + t, t)
            return carry

        lax.fori_loop(0, NCHUNK // NSET, triple, 0)
        wait_gathers(0)
        wait_scatter(0)
        compute(0)
        scatter(0)
        wait_scatter(1)
        wait_scatter(2)
        wait_scatter(0)
        plsc.subcore_barrier()
        pltpu.sync_copy(agg_sh.at[pl.ds(base_row, RPT)],
                        out_hbm.at[cid, pl.ds(base_row, RPT)])

        @pl.when(sid == NS - 1)
        def _():
            pltpu.sync_copy(agg_sh.at[pl.ds(NS * RPT, TAIL)],
                            out_hbm.at[cid, pl.ds(NS * RPT, TAIL)])

    return k(src, dst, a_tab, b_tab)


def _tail_body(agg_ref, t0_ref, wn_ref, bn_ref, w2_ref, b2_ref,
               wl_ref, bl_ref, rm_ref, rv_ref, g_ref, bb_ref,
               w4_ref, b4_ref, out_ref):
    agg = agg_ref[0] + agg_ref[1]
    h = jnp.maximum(_dot_nt(agg, wn_ref[...]) + bn_ref[...], 0.0)
    t = t0_ref[...] + _dot_nt(h, w2_ref[...]) + b2_ref[...]
    for j in range(4):
        z = _dot_nt(t, wl_ref[j]) + bl_ref[j]
        scale = jax.lax.rsqrt(rv_ref[j] + 1e-5) * g_ref[j]
        t = jnp.maximum((z - rm_ref[j]) * scale + bb_ref[j], 0.0)
    out_ref[...] = _dot_nt(t, w4_ref[...]) + b4_ref[...]


def _tail(aggp, t0, wn_t, bn, w2_t, b2, wl_t, bl, rm, rv, g, bb, w4_t, b4):
    row = pl.BlockSpec((BM, D), lambda i: (i, 0))
    aggspec = pl.BlockSpec((NC, BM, D), lambda i: (0, i, 0))
    wspec = pl.BlockSpec((D, D), lambda i: (0, 0))
    bspec = pl.BlockSpec((1, D), lambda i: (0, 0))
    wlspec = pl.BlockSpec((4, D, D), lambda i: (0, 0, 0))
    blspec = pl.BlockSpec((4, 1, D), lambda i: (0, 0, 0))
    return pl.pallas_call(
        _tail_body,
        grid=(GRID,),
        in_specs=[aggspec, row, wspec, bspec, wspec, bspec,
                  wlspec, blspec, blspec, blspec, blspec, blspec,
                  wspec, bspec],
        out_specs=row,
        out_shape=jax.ShapeDtypeStruct((N, D), jnp.float32),
    )(aggp, t0, wn_t, bn.reshape(1, D), w2_t, b2.reshape(1, D),
      wl_t, bl.reshape(4, 1, D), rm.reshape(4, 1, D), rv.reshape(4, 1, D),
      g.reshape(4, 1, D), bb.reshape(4, 1, D), w4_t, b4.reshape(1, D))


def kernel(L_x_, L_edge_index_, L_self_modules_edge_lin_parameters_weight_, L_self_modules_edge_lin_parameters_bias_, L_self_modules_cat_lin1_parameters_weight_, L_self_modules_cat_lin1_parameters_bias_, L_self_modules_cat_lin2_parameters_weight_, L_self_modules_cat_lin2_parameters_bias_, L_self_modules_node_mlp_modules_lins_modules_0_parameters_weight_, L_self_modules_node_mlp_modules_lins_modules_0_parameters_bias_, L_self_modules_final_mlp_modules_lins_modules_0_parameters_weight_, L_self_modules_final_mlp_modules_lins_modules_0_parameters_bias_, L_self_modules_final_mlp_modules_lins_modules_1_parameters_weight_, L_self_modules_final_mlp_modules_lins_modules_1_parameters_bias_, L_self_modules_final_mlp_modules_lins_modules_2_parameters_weight_, L_self_modules_final_mlp_modules_lins_modules_2_parameters_bias_, L_self_modules_final_mlp_modules_lins_modules_3_parameters_weight_, L_self_modules_final_mlp_modules_lins_modules_3_parameters_bias_, L_self_modules_final_mlp_modules_lins_modules_4_parameters_weight_, L_self_modules_final_mlp_modules_lins_modules_4_parameters_bias_, L_self_modules_final_mlp_modules_norms_modules_0_modules_module_buffers_running_mean_, L_self_modules_final_mlp_modules_norms_modules_0_modules_module_buffers_running_var_, L_self_modules_final_mlp_modules_norms_modules_0_modules_module_parameters_weight_, L_self_modules_final_mlp_modules_norms_modules_0_modules_module_parameters_bias_, L_self_modules_final_mlp_modules_norms_modules_1_modules_module_buffers_running_mean_, L_self_modules_final_mlp_modules_norms_modules_1_modules_module_buffers_running_var_, L_self_modules_final_mlp_modules_norms_modules_1_modules_module_parameters_weight_, L_self_modules_final_mlp_modules_norms_modules_1_modules_module_parameters_bias_, L_self_modules_final_mlp_modules_norms_modules_2_modules_module_buffers_running_mean_, L_self_modules_final_mlp_modules_norms_modules_2_modules_module_buffers_running_var_, L_self_modules_final_mlp_modules_norms_modules_2_modules_module_parameters_weight_, L_self_modules_final_mlp_modules_norms_modules_2_modules_module_parameters_bias_, L_self_modules_final_mlp_modules_norms_modules_3_modules_module_buffers_running_mean_, L_self_modules_final_mlp_modules_norms_modules_3_modules_module_buffers_running_var_, L_self_modules_final_mlp_modules_norms_modules_3_modules_module_parameters_weight_, L_self_modules_final_mlp_modules_norms_modules_3_modules_module_parameters_bias_):
    x = L_x_
    ei = L_edge_index_
    src = ei[0].astype(jnp.int32)
    dst = ei[1].astype(jnp.int32)
    we = L_self_modules_edge_lin_parameters_weight_
    be = L_self_modules_edge_lin_parameters_bias_

    a_tab, b_tab, t0 = _pre(
        x, we, be,
        L_self_modules_cat_lin1_parameters_weight_,
        L_self_modules_cat_lin1_parameters_bias_)

    aggp = _edge_agg(src, dst, a_tab, b_tab)

    wl_t = jnp.stack([
        L_self_modules_final_mlp_modules_lins_modules_0_parameters_weight_,
        L_self_modules_final_mlp_modules_lins_modules_1_parameters_weight_,
        L_self_modules_final_mlp_modules_lins_modules_2_parameters_weight_,
        L_self_modules_final_mlp_modules_lins_modules_3_parameters_weight_,
    ])
    bl = jnp.stack([
        L_self_modules_final_mlp_modules_lins_modules_0_parameters_bias_,
        L_self_modules_final_mlp_modules_lins_modules_1_parameters_bias_,
        L_self_modules_final_mlp_modules_lins_modules_2_parameters_bias_,
        L_self_modules_final_mlp_modules_lins_modules_3_parameters_bias_,
    ])
    rm = jnp.stack([
        L_self_modules_final_mlp_modules_norms_modules_0_modules_module_buffers_running_mean_,
        L_self_modules_final_mlp_modules_norms_modules_1_modules_module_buffers_running_mean_,
        L_self_modules_final_mlp_modules_norms_modules_2_modules_module_buffers_running_mean_,
        L_self_modules_final_mlp_modules_norms_modules_3_modules_module_buffers_running_mean_,
    ])
    rv = jnp.stack([
        L_self_modules_final_mlp_modules_norms_modules_0_modules_module_buffers_running_var_,
        L_self_modules_final_mlp_modules_norms_modules_1_modules_module_buffers_running_var_,
        L_self_modules_final_mlp_modules_norms_modules_2_modules_module_buffers_running_var_,
        L_self_modules_final_mlp_modules_norms_modules_3_modules_module_buffers_running_var_,
    ])
    g = jnp.stack([
        L_self_modules_final_mlp_modules_norms_modules_0_modules_module_parameters_weight_,
        L_self_modules_final_mlp_modules_norms_modules_1_modules_module_parameters_weight_,
        L_self_modules_final_mlp_modules_norms_modules_2_modules_module_parameters_weight_,
        L_self_modules_final_mlp_modules_norms_modules_3_modules_module_parameters_weight_,
    ])
    bb = jnp.stack([
        L_self_modules_final_mlp_modules_norms_modules_0_modules_module_parameters_bias_,
        L_self_modules_final_mlp_modules_norms_modules_1_modules_module_parameters_bias_,
        L_self_modules_final_mlp_modules_norms_modules_2_modules_module_parameters_bias_,
        L_self_modules_final_mlp_modules_norms_modules_3_modules_module_parameters_bias_,
    ])

    return _tail(
        aggp, t0,
        L_self_modules_node_mlp_modules_lins_modules_0_parameters_weight_,
        L_self_modules_node_mlp_modules_lins_modules_0_parameters_bias_,
        L_self_modules_cat_lin2_parameters_weight_,
        L_self_modules_cat_lin2_parameters_bias_,
        wl_t, bl, rm, rv, g, bb,
        L_self_modules_final_mlp_modules_lins_modules_4_parameters_weight_,
        L_self_modules_final_mlp_modules_lins_modules_4_parameters_bias_)
